# baseline (speedup 1.0000x reference)
import functools

import jax
import jax.numpy as jnp
from jax import lax
from jax.experimental import pallas as pl
from jax.experimental.pallas import tpu as pltpu
from jax.experimental.pallas import tpu_sc as plsc

_VOCAB = 100000
_EMBD = 128
_CTX = 10
_HID = 512

_BN = 1280
_NCF = 54
_TAIL = 160
_T0 = _NCF * _BN + _TAIL
_NBUF = 6

_SCW = 32
_SCB = 32
_SCC = 30
_SCN = _SCW * _SCB * _SCC


def _sc_gather(idx, emb):
    n = idx.shape[0]
    mesh = plsc.VectorSubcoreMesh(core_axis_name="c", subcore_axis_name="s")

    @functools.partial(
        pl.kernel,
        out_type=jax.ShapeDtypeStruct((n, _EMBD), jnp.float32),
        mesh=mesh,
        scratch_types=[
            pltpu.VMEM((n,), jnp.int32),
            pltpu.VMEM((n, _EMBD), jnp.float32),
            pltpu.SemaphoreType.DMA,
        ],
    )
    def k(idx_hbm, emb_hbm, out_hbm, idx_v, rows_v, sem):
        c = lax.axis_index("c")
        s = lax.axis_index("s")

        @pl.when(jnp.logical_and(c == 0, s == 0))
        def _():
            pltpu.sync_copy(idx_hbm, idx_v)
            pltpu.async_copy(emb_hbm.at[idx_v], rows_v, sem).wait()
            pltpu.sync_copy(rows_v, out_hbm)

    return k(idx, emb)


def _hid_body(e_ref, w1_ref, b1_ref, out_ref):
    h = jnp.dot(e_ref[...], w1_ref[...], preferred_element_type=jnp.float32)
    out_ref[...] = jnp.maximum(h + b1_ref[...], 0.0)


def _tc_hid(embedded, W1, b1_row):
    return pl.pallas_call(
        _hid_body,
        out_shape=jax.ShapeDtypeStruct((1, _HID), jnp.float32),
    )(embedded, W1, b1_row)


def _chunk_copy(w2t_any, buf_s, sems, c, j):
    return pltpu.make_async_copy(
        w2t_any.at[pl.ds(c * _BN, _BN), :],
        buf_s.at[j],
        sems.at[j],
    )


def _dotT(hid, chunk):
    return lax.dot_general(hid, chunk, (((1,), (1,)), ((), ())),
                           preferred_element_type=jnp.float32)


def _tc_body(hid_ref, w2t_any, out_ref, buf_s, tail_s, sems, tail_sem):
    for j in range(_NBUF):
        _chunk_copy(w2t_any, buf_s, sems, j, j).start()
    pltpu.make_async_copy(
        w2t_any.at[pl.ds(_NCF * _BN, _TAIL), :], tail_s, tail_sem).start()

    def step(s, carry):
        for j in range(_NBUF):
            c = _NBUF * s + j
            _chunk_copy(w2t_any, buf_s, sems, c, j).wait()
            out_ref[:, pl.ds(c * _BN, _BN)] = _dotT(hid_ref[...], buf_s[j])

            @pl.when(c + _NBUF < _NCF)
            def _():
                _chunk_copy(w2t_any, buf_s, sems, c + _NBUF, j).start()
        return carry

    nfull = _NCF // _NBUF
    lax.fori_loop(0, nfull, step, 0)
    for c in range(nfull * _NBUF, _NCF):
        j = c % _NBUF
        _chunk_copy(w2t_any, buf_s, sems, c, j).wait()
        out_ref[:, pl.ds(c * _BN, _BN)] = _dotT(hid_ref[...], buf_s[j])

    pltpu.make_async_copy(
        w2t_any.at[pl.ds(_NCF * _BN, _TAIL), :], tail_s, tail_sem).wait()
    out_ref[:, pl.ds(_NCF * _BN, _TAIL)] = _dotT(hid_ref[...], tail_s[...])


def _tc_matvec(hid, W2T):
    return pl.pallas_call(
        _tc_body,
        in_specs=[
            pl.BlockSpec(memory_space=pltpu.VMEM),
            pl.BlockSpec(memory_space=pl.ANY),
        ],
        out_specs=pl.BlockSpec(memory_space=pltpu.VMEM),
        out_shape=jax.ShapeDtypeStruct((1, _T0), jnp.float32),
        scratch_shapes=[
            pltpu.VMEM((_NBUF, _BN, _HID), jnp.float32),
            pltpu.VMEM((_TAIL, _HID), jnp.float32),
            pltpu.SemaphoreType.DMA((_NBUF,)),
            pltpu.SemaphoreType.DMA,
        ],
    )(hid, W2T)


def _lane_perm(x, idx):
    return lax.gather(
        x, idx.reshape(16, 1),
        lax.GatherDimensionNumbers(offset_dims=(), collapsed_slice_dims=(0,),
                                   start_index_map=(0,)),
        slice_sizes=(1,), mode=lax.GatherScatterMode.PROMISE_IN_BOUNDS)

def _sc_matvec(W2T, hid_flat):
    mesh = plsc.VectorSubcoreMesh(core_axis_name="c", subcore_axis_name="s")

    @functools.partial(
        pl.kernel,
        out_type=jax.ShapeDtypeStruct((_SCN,), jnp.float32),
        mesh=mesh,
        scratch_types=[
            pltpu.VMEM((2, _SCB, _HID), jnp.float32),
            pltpu.VMEM((_HID,), jnp.float32),
            pltpu.VMEM((_SCB * _SCC,), jnp.float32),
            pltpu.SemaphoreType.DMA((2,)),
        ],
    )
    def k(w2t_hbm, hid_hbm, out_hbm, buf_v, hid_v, out_v, sems):
        wid = lax.axis_index("s") * 2 + lax.axis_index("c")
        rbase = _T0 + wid * (_SCB * _SCC)
        obase = wid * (_SCB * _SCC)

        def copy(c, j):
            return pltpu.make_async_copy(
                w2t_hbm.at[pl.ds(rbase + c * _SCB, _SCB), :], buf_v.at[j],
                sems.at[j])

        copy(0, 0).start()
        pltpu.sync_copy(hid_hbm, hid_v)
        h = [hid_v[pl.ds(16 * j, 16)] for j in range(_HID // 16)]
        lanes = lax.iota(jnp.int32, 16)

        def chunk_body(c, j2):
            for g in range(_SCB // 16):
                res = jnp.zeros((16,), jnp.float32)
                for r in range(16):
                    row = 16 * g + r
                    acc = h[0] * buf_v[j2, row, pl.ds(0, 16)]
                    for q in range(1, _HID // 16):
                        acc = acc + h[q] * buf_v[j2, row, pl.ds(16 * q, 16)]
                    for st in (1, 2, 4, 8):
                        acc = acc + _lane_perm(acc, lanes ^ st)
                    res = jnp.where(lanes == r, acc, res)
                out_v[pl.ds(c * _SCB + 16 * g, 16)] = res

        def step(s, carry):
            for j2 in range(2):
                c = 2 * s + j2
                copy(c, j2).wait()

                @pl.when(c + 1 < _SCC)
                def _():
                    copy(c + 1, (j2 + 1) % 2).start()

                chunk_body(c, j2)
            return carry

        lax.fori_loop(0, _SCC // 2, step, 0)
        pltpu.sync_copy(out_v, out_hbm.at[pl.ds(obase, _SCB * _SCC)])

    return k(W2T, hid_flat)


def _combine_body(raw_ref, b2_ref, out_ref):
    t = raw_ref[...] + b2_ref[...]
    m = jnp.max(t)
    s = jnp.sum(jnp.exp(t - m))
    out_ref[...] = t - (m + jnp.log(s))


def _tc_combine(raw, b2_row):
    return pl.pallas_call(
        _combine_body,
        out_shape=jax.ShapeDtypeStruct((1, _VOCAB), jnp.float32),
    )(raw, b2_row)


def kernel(inputs, emb, W1, b1, W2, b2):
    embedded = _sc_gather(inputs, emb).reshape(1, 2 * _CTX * _EMBD)
    hid = _tc_hid(embedded, W1, b1.reshape(1, _HID))
    W2T = jnp.swapaxes(W2, 0, 1)
    raw_tc = _tc_matvec(hid, W2T)
    raw_sc = _sc_matvec(W2T, hid.reshape(_HID))
    raw = jnp.concatenate([raw_tc[0], raw_sc]).reshape(1, _VOCAB)
    return _tc_combine(raw, b2.reshape(1, _VOCAB))


# speedup vs baseline: 1.2863x; 1.2863x over previous
import functools

import jax
import jax.numpy as jnp
from jax import lax
from jax.experimental import pallas as pl
from jax.experimental.pallas import tpu as pltpu
from jax.experimental.pallas import tpu_sc as plsc

_VOCAB = 100000
_EMBD = 128
_CTX = 10
_HID = 512

_BN = 1280
_NCF = 49
_TAIL = 416
_T0 = _NCF * _BN + _TAIL
_NBUF = 6

_SCW = 32
_SCB = 64
_SCC = 18
_SCN = _SCW * _SCB * _SCC


def _sc_gather(idx, emb):
    n = idx.shape[0]
    mesh = plsc.VectorSubcoreMesh(core_axis_name="c", subcore_axis_name="s")

    @functools.partial(
        pl.kernel,
        out_type=jax.ShapeDtypeStruct((n, _EMBD), jnp.float32),
        mesh=mesh,
        scratch_types=[
            pltpu.VMEM((n,), jnp.int32),
            pltpu.VMEM((n, _EMBD), jnp.float32),
            pltpu.SemaphoreType.DMA,
        ],
    )
    def k(idx_hbm, emb_hbm, out_hbm, idx_v, rows_v, sem):
        c = lax.axis_index("c")
        s = lax.axis_index("s")

        @pl.when(jnp.logical_and(c == 0, s == 0))
        def _():
            pltpu.sync_copy(idx_hbm, idx_v)
            pltpu.async_copy(emb_hbm.at[idx_v], rows_v, sem).wait()
            pltpu.sync_copy(rows_v, out_hbm)

    return k(idx, emb)


def _hid_body(e_ref, w1_ref, b1_ref, out_ref):
    h = jnp.dot(e_ref[...], w1_ref[...], preferred_element_type=jnp.float32)
    out_ref[...] = jnp.maximum(h + b1_ref[...], 0.0)


def _tc_hid(embedded, W1, b1_row):
    return pl.pallas_call(
        _hid_body,
        out_shape=jax.ShapeDtypeStruct((1, _HID), jnp.float32),
    )(embedded, W1, b1_row)


def _chunk_copy(w2t_any, buf_s, sems, c, j):
    return pltpu.make_async_copy(
        w2t_any.at[pl.ds(c * _BN, _BN), :],
        buf_s.at[j],
        sems.at[j],
    )


def _dotT(hid, chunk):
    return lax.dot_general(hid, chunk, (((1,), (1,)), ((), ())),
                           preferred_element_type=jnp.float32)


def _tc_body(hid_ref, w2t_any, out_ref, buf_s, tail_s, sems, tail_sem):
    for j in range(_NBUF):
        _chunk_copy(w2t_any, buf_s, sems, j, j).start()
    pltpu.make_async_copy(
        w2t_any.at[pl.ds(_NCF * _BN, _TAIL), :], tail_s, tail_sem).start()

    def step(s, carry):
        for j in range(_NBUF):
            c = _NBUF * s + j
            _chunk_copy(w2t_any, buf_s, sems, c, j).wait()
            out_ref[:, pl.ds(c * _BN, _BN)] = _dotT(hid_ref[...], buf_s[j])

            @pl.when(c + _NBUF < _NCF)
            def _():
                _chunk_copy(w2t_any, buf_s, sems, c + _NBUF, j).start()
        return carry

    nfull = _NCF // _NBUF
    lax.fori_loop(0, nfull, step, 0)
    for c in range(nfull * _NBUF, _NCF):
        j = c % _NBUF
        _chunk_copy(w2t_any, buf_s, sems, c, j).wait()
        out_ref[:, pl.ds(c * _BN, _BN)] = _dotT(hid_ref[...], buf_s[j])

    pltpu.make_async_copy(
        w2t_any.at[pl.ds(_NCF * _BN, _TAIL), :], tail_s, tail_sem).wait()
    out_ref[:, pl.ds(_NCF * _BN, _TAIL)] = _dotT(hid_ref[...], tail_s[...])


def _tc_matvec(hid, W2T):
    return pl.pallas_call(
        _tc_body,
        in_specs=[
            pl.BlockSpec(memory_space=pltpu.VMEM),
            pl.BlockSpec(memory_space=pl.ANY),
        ],
        out_specs=pl.BlockSpec(memory_space=pltpu.VMEM),
        out_shape=jax.ShapeDtypeStruct((1, _T0), jnp.float32),
        scratch_shapes=[
            pltpu.VMEM((_NBUF, _BN, _HID), jnp.float32),
            pltpu.VMEM((_TAIL, _HID), jnp.float32),
            pltpu.SemaphoreType.DMA((_NBUF,)),
            pltpu.SemaphoreType.DMA,
        ],
    )(hid, W2T)


def _lane_perm(x, idx):
    return lax.gather(
        x, idx.reshape(16, 1),
        lax.GatherDimensionNumbers(offset_dims=(), collapsed_slice_dims=(0,),
                                   start_index_map=(0,)),
        slice_sizes=(1,), mode=lax.GatherScatterMode.PROMISE_IN_BOUNDS)

def _sc_matvec(W2T, hid_flat):
    mesh = plsc.VectorSubcoreMesh(core_axis_name="c", subcore_axis_name="s")

    @functools.partial(
        pl.kernel,
        out_type=jax.ShapeDtypeStruct((_SCN,), jnp.float32),
        mesh=mesh,
        scratch_types=[
            pltpu.VMEM((2, _SCB, _HID), jnp.float32),
            pltpu.VMEM((_HID,), jnp.float32),
            pltpu.VMEM((_SCB * _SCC,), jnp.float32),
            pltpu.SemaphoreType.DMA((2,)),
        ],
    )
    def k(w2t_hbm, hid_hbm, out_hbm, buf_v, hid_v, out_v, sems):
        wid = lax.axis_index("s") * 2 + lax.axis_index("c")
        rbase = _T0 + wid * (_SCB * _SCC)
        obase = wid * (_SCB * _SCC)

        def copy(c, j):
            return pltpu.make_async_copy(
                w2t_hbm.at[pl.ds(rbase + c * _SCB, _SCB), :], buf_v.at[j],
                sems.at[j])

        copy(0, 0).start()
        pltpu.sync_copy(hid_hbm, hid_v)
        h = [hid_v[pl.ds(16 * j, 16)] for j in range(_HID // 16)]
        lanes = lax.iota(jnp.int32, 16)

        def chunk_body(c, j2):
            def group(g, carry):
                accs = []
                for r in range(16):
                    row = 16 * g + r
                    acc = h[0] * buf_v[j2, row, pl.ds(0, 16)]
                    for q in range(1, _HID // 16):
                        acc = acc + h[q] * buf_v[j2, row, pl.ds(16 * q, 16)]
                    accs.append(acc)
                for k in range(4):
                    nxt = []
                    for i in range(len(accs) // 2):
                        a, b = accs[2 * i], accs[2 * i + 1]
                        pa = a + _lane_perm(a, lanes ^ (1 << k))
                        pb = b + _lane_perm(b, lanes ^ (1 << k))
                        nxt.append(jnp.where((lanes & (1 << k)) == 0, pa, pb))
                    accs = nxt
                out_v[pl.ds(c * _SCB + 16 * g, 16)] = accs[0]
                return carry

            lax.fori_loop(0, _SCB // 16, group, 0)

        def step(s, carry):
            for j2 in range(2):
                c = 2 * s + j2
                copy(c, j2).wait()

                @pl.when(c + 1 < _SCC)
                def _():
                    copy(c + 1, (j2 + 1) % 2).start()

                chunk_body(c, j2)
            return carry

        lax.fori_loop(0, _SCC // 2, step, 0)
        pltpu.sync_copy(out_v, out_hbm.at[pl.ds(obase, _SCB * _SCC)])

    return k(W2T, hid_flat)


def _combine_body(raw_ref, b2_ref, out_ref):
    t = raw_ref[...] + b2_ref[...]
    m = jnp.max(t)
    s = jnp.sum(jnp.exp(t - m))
    out_ref[...] = t - (m + jnp.log(s))


def _tc_combine(raw, b2_row):
    return pl.pallas_call(
        _combine_body,
        out_shape=jax.ShapeDtypeStruct((1, _VOCAB), jnp.float32),
    )(raw, b2_row)


def kernel(inputs, emb, W1, b1, W2, b2):
    embedded = _sc_gather(inputs, emb).reshape(1, 2 * _CTX * _EMBD)
    hid = _tc_hid(embedded, W1, b1.reshape(1, _HID))
    W2T = jnp.swapaxes(W2, 0, 1)
    raw_tc = _tc_matvec(hid, W2T)
    raw_sc = _sc_matvec(W2T, hid.reshape(_HID))
    raw = jnp.concatenate([raw_tc[0], raw_sc]).reshape(1, _VOCAB)
    return _tc_combine(raw, b2.reshape(1, _VOCAB))


# speedup vs baseline: 1.4542x; 1.1305x over previous
import functools

import jax
import jax.numpy as jnp
from jax import lax
from jax.experimental import pallas as pl
from jax.experimental.pallas import tpu as pltpu
from jax.experimental.pallas import tpu_sc as plsc

_VOCAB = 100000
_EMBD = 128
_CTX = 10
_HID = 512

_BN = 1280
_NCF = 55
_TAIL = 928
_T0 = _NCF * _BN + _TAIL
_NBUF = 6

_SCW = 32
_SCB = 64
_SCC = 14
_SCN = _SCW * _SCB * _SCC


def _sc_gather(idx, emb):
    n = idx.shape[0]
    mesh = plsc.VectorSubcoreMesh(core_axis_name="c", subcore_axis_name="s")

    @functools.partial(
        pl.kernel,
        out_type=jax.ShapeDtypeStruct((n, _EMBD), jnp.float32),
        mesh=mesh,
        scratch_types=[
            pltpu.VMEM((n,), jnp.int32),
            pltpu.VMEM((n, _EMBD), jnp.float32),
            pltpu.SemaphoreType.DMA,
        ],
    )
    def k(idx_hbm, emb_hbm, out_hbm, idx_v, rows_v, sem):
        c = lax.axis_index("c")
        s = lax.axis_index("s")

        @pl.when(jnp.logical_and(c == 0, s == 0))
        def _():
            pltpu.sync_copy(idx_hbm, idx_v)
            pltpu.async_copy(emb_hbm.at[idx_v], rows_v, sem).wait()
            pltpu.sync_copy(rows_v, out_hbm)

    return k(idx, emb)


def _hid_body(idx_ref, emb_any, w1_ref, b1_ref, out_ref, rows_v, sems):
    for t in range(2 * _CTX):
        pltpu.make_async_copy(
            emb_any.at[pl.ds(idx_ref[t], 1), :],
            rows_v.at[pl.ds(t, 1), :],
            sems.at[t],
        ).start()
    acc = b1_ref[...]
    for t in range(2 * _CTX):
        pltpu.make_async_copy(
            emb_any.at[pl.ds(idx_ref[t], 1), :],
            rows_v.at[pl.ds(t, 1), :],
            sems.at[t],
        ).wait()
        acc = acc + jnp.dot(rows_v[pl.ds(t, 1), :], w1_ref[t],
                            preferred_element_type=jnp.float32)
    out_ref[...] = jnp.maximum(acc, 0.0)


def _tc_hid(idx, emb, W1s, b1_row):
    return pl.pallas_call(
        _hid_body,
        in_specs=[
            pl.BlockSpec(memory_space=pltpu.SMEM),
            pl.BlockSpec(memory_space=pl.ANY),
            pl.BlockSpec(memory_space=pltpu.VMEM),
            pl.BlockSpec(memory_space=pltpu.VMEM),
        ],
        out_specs=pl.BlockSpec(memory_space=pltpu.VMEM),
        out_shape=jax.ShapeDtypeStruct((1, _HID), jnp.float32),
        scratch_shapes=[
            pltpu.VMEM((2 * _CTX, _EMBD), jnp.float32),
            pltpu.SemaphoreType.DMA((2 * _CTX,)),
        ],
    )(idx, emb, W1s, b1_row)


def _chunk_copy(w2t_any, buf_s, sems, c, j):
    return pltpu.make_async_copy(
        w2t_any.at[pl.ds(c * _BN, _BN), :],
        buf_s.at[j],
        sems.at[j],
    )


def _dotT(hid, chunk):
    return lax.dot_general(hid, chunk, (((1,), (1,)), ((), ())),
                           preferred_element_type=jnp.float32)


def _tc_body(hid_ref, w2t_any, out_ref, buf_s, tail_s, sems, tail_sem):
    for j in range(_NBUF):
        _chunk_copy(w2t_any, buf_s, sems, j, j).start()
    pltpu.make_async_copy(
        w2t_any.at[pl.ds(_NCF * _BN, _TAIL), :], tail_s, tail_sem).start()

    def step(s, carry):
        for j in range(_NBUF):
            c = _NBUF * s + j
            _chunk_copy(w2t_any, buf_s, sems, c, j).wait()
            out_ref[:, pl.ds(c * _BN, _BN)] = _dotT(hid_ref[...], buf_s[j])

            @pl.when(c + _NBUF < _NCF)
            def _():
                _chunk_copy(w2t_any, buf_s, sems, c + _NBUF, j).start()
        return carry

    nfull = _NCF // _NBUF
    lax.fori_loop(0, nfull, step, 0)
    for c in range(nfull * _NBUF, _NCF):
        j = c % _NBUF
        _chunk_copy(w2t_any, buf_s, sems, c, j).wait()
        out_ref[:, pl.ds(c * _BN, _BN)] = _dotT(hid_ref[...], buf_s[j])

    pltpu.make_async_copy(
        w2t_any.at[pl.ds(_NCF * _BN, _TAIL), :], tail_s, tail_sem).wait()
    out_ref[:, pl.ds(_NCF * _BN, _TAIL)] = _dotT(hid_ref[...], tail_s[...])


def _tc_matvec(hid, W2T):
    return pl.pallas_call(
        _tc_body,
        in_specs=[
            pl.BlockSpec(memory_space=pltpu.VMEM),
            pl.BlockSpec(memory_space=pl.ANY),
        ],
        out_specs=pl.BlockSpec(memory_space=pltpu.VMEM),
        out_shape=jax.ShapeDtypeStruct((1, _T0), jnp.float32),
        scratch_shapes=[
            pltpu.VMEM((_NBUF, _BN, _HID), jnp.float32),
            pltpu.VMEM((_TAIL, _HID), jnp.float32),
            pltpu.SemaphoreType.DMA((_NBUF,)),
            pltpu.SemaphoreType.DMA,
        ],
    )(hid, W2T)


def _lane_perm(x, idx):
    return lax.gather(
        x, idx.reshape(16, 1),
        lax.GatherDimensionNumbers(offset_dims=(), collapsed_slice_dims=(0,),
                                   start_index_map=(0,)),
        slice_sizes=(1,), mode=lax.GatherScatterMode.PROMISE_IN_BOUNDS)

def _sc_matvec(W2T, hid_flat):
    mesh = plsc.VectorSubcoreMesh(core_axis_name="c", subcore_axis_name="s")

    @functools.partial(
        pl.kernel,
        out_type=jax.ShapeDtypeStruct((_SCN,), jnp.float32),
        mesh=mesh,
        scratch_types=[
            pltpu.VMEM((2, _SCB, _HID), jnp.float32),
            pltpu.VMEM((_HID,), jnp.float32),
            pltpu.VMEM((_SCB * _SCC,), jnp.float32),
            pltpu.SemaphoreType.DMA((2,)),
        ],
    )
    def k(w2t_hbm, hid_hbm, out_hbm, buf_v, hid_v, out_v, sems):
        wid = lax.axis_index("s") * 2 + lax.axis_index("c")
        rbase = _T0 + wid * (_SCB * _SCC)
        obase = wid * (_SCB * _SCC)

        def copy(c, j):
            return pltpu.make_async_copy(
                w2t_hbm.at[pl.ds(rbase + c * _SCB, _SCB), :], buf_v.at[j],
                sems.at[j])

        copy(0, 0).start()
        pltpu.sync_copy(hid_hbm, hid_v)
        h = [hid_v[pl.ds(16 * j, 16)] for j in range(_HID // 16)]
        lanes = lax.iota(jnp.int32, 16)

        def chunk_body(c, j2):
            def group(g, carry):
                accs = []
                for r in range(16):
                    row = 16 * g + r
                    acc = h[0] * buf_v[j2, row, pl.ds(0, 16)]
                    for q in range(1, _HID // 16):
                        acc = acc + h[q] * buf_v[j2, row, pl.ds(16 * q, 16)]
                    accs.append(acc)
                for k in range(4):
                    nxt = []
                    for i in range(len(accs) // 2):
                        a, b = accs[2 * i], accs[2 * i + 1]
                        pa = a + _lane_perm(a, lanes ^ (1 << k))
                        pb = b + _lane_perm(b, lanes ^ (1 << k))
                        nxt.append(jnp.where((lanes & (1 << k)) == 0, pa, pb))
                    accs = nxt
                out_v[pl.ds(c * _SCB + 16 * g, 16)] = accs[0]
                return carry

            lax.fori_loop(0, _SCB // 16, group, 0)

        def step(s, carry):
            for j2 in range(2):
                c = 2 * s + j2
                copy(c, j2).wait()

                @pl.when(c + 1 < _SCC)
                def _():
                    copy(c + 1, (j2 + 1) % 2).start()

                chunk_body(c, j2)
            return carry

        lax.fori_loop(0, _SCC // 2, step, 0)
        pltpu.sync_copy(out_v, out_hbm.at[pl.ds(obase, _SCB * _SCC)])

    return k(W2T, hid_flat)


def _combine_body(raw_ref, b2_ref, out_ref):
    t = raw_ref[...] + b2_ref[...]
    m = jnp.max(t)
    s = jnp.sum(jnp.exp(t - m))
    out_ref[...] = t - (m + jnp.log(s))


def _tc_combine(raw, b2_row):
    return pl.pallas_call(
        _combine_body,
        out_shape=jax.ShapeDtypeStruct((1, _VOCAB), jnp.float32),
    )(raw, b2_row)


def kernel(inputs, emb, W1, b1, W2, b2):
    hid = _tc_hid(inputs, emb, W1.reshape(2 * _CTX, _EMBD, _HID),
                  b1.reshape(1, _HID))
    W2T = jnp.swapaxes(W2, 0, 1)
    raw_tc = _tc_matvec(hid, W2T)
    raw_sc = _sc_matvec(W2T, hid.reshape(_HID))
    raw = jnp.concatenate([raw_tc[0], raw_sc]).reshape(1, _VOCAB)
    return _tc_combine(raw, b2.reshape(1, _VOCAB))


# speedup vs baseline: 1.4612x; 1.0048x over previous
import functools

import jax
import jax.numpy as jnp
from jax import lax
from jax.experimental import pallas as pl
from jax.experimental.pallas import tpu as pltpu
from jax.experimental.pallas import tpu_sc as plsc

_VOCAB = 100000
_EMBD = 128
_CTX = 10
_HID = 512

_BN = 1280
_NCF = 58
_TAIL = 1184
_T0 = _NCF * _BN + _TAIL
_NBUF = 6

_SCW = 32
_SCB = 64
_SCC = 12
_SCN = _SCW * _SCB * _SCC


def _sc_gather(idx, emb):
    n = idx.shape[0]
    mesh = plsc.VectorSubcoreMesh(core_axis_name="c", subcore_axis_name="s")

    @functools.partial(
        pl.kernel,
        out_type=jax.ShapeDtypeStruct((n, _EMBD), jnp.float32),
        mesh=mesh,
        scratch_types=[
            pltpu.VMEM((n,), jnp.int32),
            pltpu.VMEM((n, _EMBD), jnp.float32),
            pltpu.SemaphoreType.DMA,
        ],
    )
    def k(idx_hbm, emb_hbm, out_hbm, idx_v, rows_v, sem):
        c = lax.axis_index("c")
        s = lax.axis_index("s")

        @pl.when(jnp.logical_and(c == 0, s == 0))
        def _():
            pltpu.sync_copy(idx_hbm, idx_v)
            pltpu.async_copy(emb_hbm.at[idx_v], rows_v, sem).wait()
            pltpu.sync_copy(rows_v, out_hbm)

    return k(idx, emb)


def _hid_body(idx_ref, emb_any, w1_ref, b1_ref, out_ref, rows_v, sems):
    for t in range(2 * _CTX):
        pltpu.make_async_copy(
            emb_any.at[pl.ds(idx_ref[t], 1), :],
            rows_v.at[pl.ds(t, 1), :],
            sems.at[t],
        ).start()
    acc = b1_ref[...]
    for t in range(2 * _CTX):
        pltpu.make_async_copy(
            emb_any.at[pl.ds(idx_ref[t], 1), :],
            rows_v.at[pl.ds(t, 1), :],
            sems.at[t],
        ).wait()
        acc = acc + jnp.dot(rows_v[pl.ds(t, 1), :], w1_ref[t],
                            preferred_element_type=jnp.float32)
    out_ref[...] = jnp.maximum(acc, 0.0)


def _tc_hid(idx, emb, W1s, b1_row):
    return pl.pallas_call(
        _hid_body,
        in_specs=[
            pl.BlockSpec(memory_space=pltpu.SMEM),
            pl.BlockSpec(memory_space=pl.ANY),
            pl.BlockSpec(memory_space=pltpu.VMEM),
            pl.BlockSpec(memory_space=pltpu.VMEM),
        ],
        out_specs=pl.BlockSpec(memory_space=pltpu.VMEM),
        out_shape=jax.ShapeDtypeStruct((1, _HID), jnp.float32),
        scratch_shapes=[
            pltpu.VMEM((2 * _CTX, _EMBD), jnp.float32),
            pltpu.SemaphoreType.DMA((2 * _CTX,)),
        ],
    )(idx, emb, W1s, b1_row)


def _chunk_copy(w2t_any, buf_s, sems, c, j):
    return pltpu.make_async_copy(
        w2t_any.at[pl.ds(c * _BN, _BN), :],
        buf_s.at[j],
        sems.at[j],
    )


def _dotT(hid, chunk):
    return lax.dot_general(hid, chunk, (((1,), (1,)), ((), ())),
                           preferred_element_type=jnp.float32)


def _tc_body(hid_ref, w2t_any, out_ref, buf_s, tail_s, sems, tail_sem):
    for j in range(_NBUF):
        _chunk_copy(w2t_any, buf_s, sems, j, j).start()
    pltpu.make_async_copy(
        w2t_any.at[pl.ds(_NCF * _BN, _TAIL), :], tail_s, tail_sem).start()

    def step(s, carry):
        for j in range(_NBUF):
            c = _NBUF * s + j
            _chunk_copy(w2t_any, buf_s, sems, c, j).wait()
            out_ref[:, pl.ds(c * _BN, _BN)] = _dotT(hid_ref[...], buf_s[j])

            @pl.when(c + _NBUF < _NCF)
            def _():
                _chunk_copy(w2t_any, buf_s, sems, c + _NBUF, j).start()
        return carry

    nfull = _NCF // _NBUF
    lax.fori_loop(0, nfull, step, 0)
    for c in range(nfull * _NBUF, _NCF):
        j = c % _NBUF
        _chunk_copy(w2t_any, buf_s, sems, c, j).wait()
        out_ref[:, pl.ds(c * _BN, _BN)] = _dotT(hid_ref[...], buf_s[j])

    pltpu.make_async_copy(
        w2t_any.at[pl.ds(_NCF * _BN, _TAIL), :], tail_s, tail_sem).wait()
    out_ref[:, pl.ds(_NCF * _BN, _TAIL)] = _dotT(hid_ref[...], tail_s[...])


def _tc_matvec(hid, W2T):
    return pl.pallas_call(
        _tc_body,
        in_specs=[
            pl.BlockSpec(memory_space=pltpu.VMEM),
            pl.BlockSpec(memory_space=pl.ANY),
        ],
        out_specs=pl.BlockSpec(memory_space=pltpu.VMEM),
        out_shape=jax.ShapeDtypeStruct((1, _T0), jnp.float32),
        scratch_shapes=[
            pltpu.VMEM((_NBUF, _BN, _HID), jnp.float32),
            pltpu.VMEM((_TAIL, _HID), jnp.float32),
            pltpu.SemaphoreType.DMA((_NBUF,)),
            pltpu.SemaphoreType.DMA,
        ],
    )(hid, W2T)


def _lane_perm(x, idx):
    return lax.gather(
        x, idx.reshape(16, 1),
        lax.GatherDimensionNumbers(offset_dims=(), collapsed_slice_dims=(0,),
                                   start_index_map=(0,)),
        slice_sizes=(1,), mode=lax.GatherScatterMode.PROMISE_IN_BOUNDS)

def _sc_matvec(W2T, hid_flat):
    mesh = plsc.VectorSubcoreMesh(core_axis_name="c", subcore_axis_name="s")

    @functools.partial(
        pl.kernel,
        out_type=jax.ShapeDtypeStruct((_SCN,), jnp.float32),
        mesh=mesh,
        scratch_types=[
            pltpu.VMEM((2, _SCB, _HID), jnp.float32),
            pltpu.VMEM((_HID,), jnp.float32),
            pltpu.VMEM((_SCB * _SCC,), jnp.float32),
            pltpu.SemaphoreType.DMA((2,)),
        ],
    )
    def k(w2t_hbm, hid_hbm, out_hbm, buf_v, hid_v, out_v, sems):
        wid = lax.axis_index("s") * 2 + lax.axis_index("c")
        rbase = _T0 + wid * (_SCB * _SCC)
        obase = wid * (_SCB * _SCC)

        def copy(c, j):
            return pltpu.make_async_copy(
                w2t_hbm.at[pl.ds(rbase + c * _SCB, _SCB), :], buf_v.at[j],
                sems.at[j])

        copy(0, 0).start()
        pltpu.sync_copy(hid_hbm, hid_v)
        h = [hid_v[pl.ds(16 * j, 16)] for j in range(_HID // 16)]
        lanes = lax.iota(jnp.int32, 16)

        def chunk_body(c, j2):
            def group(g, carry):
                accs = []
                for r in range(16):
                    row = 16 * g + r
                    acc = h[0] * buf_v[j2, row, pl.ds(0, 16)]
                    for q in range(1, _HID // 16):
                        acc = acc + h[q] * buf_v[j2, row, pl.ds(16 * q, 16)]
                    accs.append(acc)
                for k in range(4):
                    nxt = []
                    for i in range(len(accs) // 2):
                        a, b = accs[2 * i], accs[2 * i + 1]
                        pa = a + _lane_perm(a, lanes ^ (1 << k))
                        pb = b + _lane_perm(b, lanes ^ (1 << k))
                        nxt.append(jnp.where((lanes & (1 << k)) == 0, pa, pb))
                    accs = nxt
                out_v[pl.ds(c * _SCB + 16 * g, 16)] = accs[0]
                return carry

            lax.fori_loop(0, _SCB // 16, group, 0)

        def step(s, carry):
            for j2 in range(2):
                c = 2 * s + j2
                copy(c, j2).wait()

                @pl.when(c + 1 < _SCC)
                def _():
                    copy(c + 1, (j2 + 1) % 2).start()

                chunk_body(c, j2)
            return carry

        lax.fori_loop(0, _SCC // 2, step, 0)
        pltpu.sync_copy(out_v, out_hbm.at[pl.ds(obase, _SCB * _SCC)])

    return k(W2T, hid_flat)


def _combine_body(raw_ref, b2_ref, out_ref):
    t = raw_ref[...] + b2_ref[...]
    m = jnp.max(t)
    s = jnp.sum(jnp.exp(t - m))
    out_ref[...] = t - (m + jnp.log(s))


def _tc_combine(raw, b2_row):
    return pl.pallas_call(
        _combine_body,
        out_shape=jax.ShapeDtypeStruct((1, _VOCAB), jnp.float32),
    )(raw, b2_row)


def kernel(inputs, emb, W1, b1, W2, b2):
    hid = _tc_hid(inputs, emb, W1.reshape(2 * _CTX, _EMBD, _HID),
                  b1.reshape(1, _HID))
    W2T = jnp.swapaxes(W2, 0, 1)
    raw_tc = _tc_matvec(hid, W2T)
    raw_sc = _sc_matvec(W2T, hid.reshape(_HID))
    raw = jnp.concatenate([raw_tc[0], raw_sc]).reshape(1, _VOCAB)
    return _tc_combine(raw, b2.reshape(1, _VOCAB))


# speedup vs baseline: 1.5308x; 1.0476x over previous
import functools

import jax
import jax.numpy as jnp
from jax import lax
from jax.experimental import pallas as pl
from jax.experimental.pallas import tpu as pltpu
from jax.experimental.pallas import tpu_sc as plsc

_VOCAB = 100000
_EMBD = 128
_CTX = 10
_HID = 512
_BN = 1280
_NC = _VOCAB // _BN
_TAIL = _VOCAB - _NC * _BN
_NBUF = 6
_NSTEP = _NC // _NBUF


def _sc_gather(idx, emb):
    n = idx.shape[0]
    mesh = plsc.VectorSubcoreMesh(core_axis_name="c", subcore_axis_name="s")

    @functools.partial(
        pl.kernel,
        out_type=jax.ShapeDtypeStruct((n, _EMBD), jnp.float32),
        mesh=mesh,
        scratch_types=[
            pltpu.VMEM((n,), jnp.int32),
            pltpu.VMEM((n, _EMBD), jnp.float32),
            pltpu.SemaphoreType.DMA,
        ],
    )
    def k(idx_hbm, emb_hbm, out_hbm, idx_v, rows_v, sem):
        c = lax.axis_index("c")
        s = lax.axis_index("s")

        @pl.when(jnp.logical_and(c == 0, s == 0))
        def _():
            pltpu.sync_copy(idx_hbm, idx_v)
            pltpu.async_copy(emb_hbm.at[idx_v], rows_v, sem).wait()
            pltpu.sync_copy(rows_v, out_hbm)

    return k(idx, emb)


def _chunk_copy(w2t_any, buf_s, sems, c, j):
    return pltpu.make_async_copy(
        w2t_any.at[pl.ds(c * _BN, _BN), :],
        buf_s.at[j],
        sems.at[j],
    )


def _dotT(hid, chunk):
    return lax.dot_general(hid, chunk, (((1,), (1,)), ((), ())),
                           preferred_element_type=jnp.float32)


def _mega_body(e_ref, w1_ref, b1_ref, b2_ref, w2t_any, out_ref,
               hid_s, out_s, buf_s, tail_s, sems, tail_sem):
    for j in range(_NBUF):
        _chunk_copy(w2t_any, buf_s, sems, j, j).start()
    pltpu.make_async_copy(
        w2t_any.at[pl.ds(_NC * _BN, _TAIL), :], tail_s, tail_sem).start()

    h = jnp.dot(e_ref[...], w1_ref[...], preferred_element_type=jnp.float32)
    hid_s[...] = jnp.maximum(h + b1_ref[...], 0.0)

    def step(s, carry):
        m0, s0 = carry
        for j in range(_NBUF):
            c = _NBUF * s + j
            _chunk_copy(w2t_any, buf_s, sems, c, j).wait()
            blk = _dotT(hid_s[...], buf_s[j])
            blk = blk + b2_ref[:, pl.ds(c * _BN, _BN)]
            out_s[:, pl.ds(c * _BN, _BN)] = blk
            m1 = jnp.maximum(m0, jnp.max(blk))
            s0 = s0 * jnp.exp(m0 - m1) + jnp.sum(jnp.exp(blk - m1))
            m0 = m1

            @pl.when(c + _NBUF < _NC)
            def _():
                _chunk_copy(w2t_any, buf_s, sems, c + _NBUF, j).start()
        return m0, s0

    m0, s0 = lax.fori_loop(
        0, _NSTEP, step, (jnp.float32(-jnp.inf), jnp.float32(0.0)))

    pltpu.make_async_copy(
        w2t_any.at[pl.ds(_NC * _BN, _TAIL), :], tail_s, tail_sem).wait()
    blk = _dotT(hid_s[...], tail_s[...])
    blk = blk + b2_ref[:, pl.ds(_NC * _BN, _TAIL)]
    out_s[:, pl.ds(_NC * _BN, _TAIL)] = blk
    m1 = jnp.maximum(m0, jnp.max(blk))
    s1 = s0 * jnp.exp(m0 - m1) + jnp.sum(jnp.exp(blk - m1))

    lse = m1 + jnp.log(s1)
    out_ref[...] = out_s[...] - lse


def _tc_mlp(embedded, W1, b1_row, W2T, b2_row):
    return pl.pallas_call(
        _mega_body,
        in_specs=[
            pl.BlockSpec(memory_space=pltpu.VMEM),
            pl.BlockSpec(memory_space=pltpu.VMEM),
            pl.BlockSpec(memory_space=pltpu.VMEM),
            pl.BlockSpec(memory_space=pltpu.VMEM),
            pl.BlockSpec(memory_space=pl.ANY),
        ],
        out_specs=pl.BlockSpec(memory_space=pltpu.VMEM),
        out_shape=jax.ShapeDtypeStruct((1, _VOCAB), jnp.float32),
        scratch_shapes=[
            pltpu.VMEM((1, _HID), jnp.float32),
            pltpu.VMEM((1, _VOCAB), jnp.float32),
            pltpu.VMEM((_NBUF, _BN, _HID), jnp.float32),
            pltpu.VMEM((_TAIL, _HID), jnp.float32),
            pltpu.SemaphoreType.DMA((_NBUF,)),
            pltpu.SemaphoreType.DMA,
        ],
    )(embedded, W1, b1_row, b2_row, W2T)


def kernel(inputs, emb, W1, b1, W2, b2):
    embedded = _sc_gather(inputs, emb).reshape(1, 2 * _CTX * _EMBD)
    return _tc_mlp(embedded, W1, b1.reshape(1, _HID), jnp.swapaxes(W2, 0, 1),
                   b2.reshape(1, _VOCAB))
